# Initial kernel scaffold; baseline (speedup 1.0000x reference)
import jax
import jax.numpy as jnp
from jax.experimental import pallas as pl


def kernel(x):
    raise NotImplementedError("write your pallas kernel here")


# speedup vs baseline: 2.1477x; 2.1477x over previous
import functools

import jax
import jax.numpy as jnp
from jax import lax
from jax.experimental import pallas as pl
from jax.experimental.pallas import tpu as pltpu
from jax.experimental.pallas import tpu_sc as plsc

W = 384
L = 16
CW = W // L
NW = 32
CH = 64


def _mirror_body(x_hbm, out_hbm, in_v, out_v):
    wid = lax.axis_index("s") * 2 + lax.axis_index("c")
    rows_total = x_hbm.shape[0]
    rows_per_w = rows_total // NW
    n_chunks = rows_per_w // CH
    w_base = wid * rows_per_w

    def chunk_body(g, carry):
        row0 = w_base + g * CH
        pltpu.sync_copy(x_hbm.at[pl.ds(row0, CH), :], in_v)

        def row_body(r, c2):
            for c in range(CW):
                v = in_v[r, pl.ds((CW - 1 - c) * L, L)]
                out_v[r, pl.ds(c * L, L)] = lax.rev(v, (0,))
            return c2

        lax.fori_loop(0, CH, row_body, 0)
        pltpu.sync_copy(out_v, out_hbm.at[pl.ds(row0, CH), :])
        return carry

    lax.fori_loop(0, n_chunks, chunk_body, 0)


def kernel(x):
    b, c, h, w = x.shape
    rows = b * c * h
    x2 = x.reshape(rows, w)
    mesh = plsc.VectorSubcoreMesh(core_axis_name="c", subcore_axis_name="s")
    run = functools.partial(
        pl.kernel,
        mesh=mesh,
        out_type=jax.ShapeDtypeStruct((rows, w), jnp.float32),
        scratch_types=[
            pltpu.VMEM((CH, W), jnp.float32),
            pltpu.VMEM((CH, W), jnp.float32),
        ],
    )(_mirror_body)
    out2 = run(x2)
    return out2.reshape(b, c, h, w)
